# baseline (speedup 1.0000x reference)
import functools

import jax
import jax.numpy as jnp
from jax.experimental import pallas as pl
from jax.experimental.pallas import tpu as pltpu

BQ = 256
BK = 256
NEG_INF = -1e30


def _flash_kernel(q_ref, k_ref, v_ref, o_ref, *, bq, bk, d):
    qi = pl.program_id(1)
    q = q_ref[0]

    def block_update(s, v, carry):
        acc, m = carry
        m_new = jnp.maximum(m, jnp.max(s, axis=-1))
        alpha = jnp.exp(m - m_new)
        p = jnp.exp(s - m_new[:, None]).astype(jnp.bfloat16)
        acc_new = acc * alpha[:, None] + jax.lax.dot_general(
            p, v, (((1,), (0,)), ((), ())),
            preferred_element_type=jnp.float32)
        return acc_new, m_new

    def body(j, carry):
        k = k_ref[0, pl.ds(j * bk, bk), :]
        v = v_ref[0, pl.ds(j * bk, bk), :]
        s = jax.lax.dot_general(
            q, k, (((1,), (1,)), ((), ())),
            preferred_element_type=jnp.float32)
        return block_update(s, v, carry)

    init = (jnp.zeros((bq, 2 * d), jnp.float32),
            jnp.full((bq,), NEG_INF, jnp.float32))
    acc, m = jax.lax.fori_loop(0, qi, body, init)

    k = k_ref[0, pl.ds(qi * bk, bk), :]
    v = v_ref[0, pl.ds(qi * bk, bk), :]
    s = jax.lax.dot_general(
        q, k, (((1,), (1,)), ((), ())),
        preferred_element_type=jnp.float32)
    row = jax.lax.broadcasted_iota(jnp.int32, (bq, bk), 0)
    col = jax.lax.broadcasted_iota(jnp.int32, (bq, bk), 1)
    s = jnp.where(col <= row, s, NEG_INF)
    acc, m = block_update(s, v, (acc, m))

    o_ref[0] = acc[:, :d] / acc[:, d:d + 1]


@jax.jit
def kernel(query, key, value):
    b, h, s, d = query.shape
    scale = d ** (-0.5)
    q = (query.reshape(b * h, s, d) * scale).astype(jnp.bfloat16)
    k = key.reshape(b * h, s, d).astype(jnp.bfloat16)
    v = value.reshape(b * h, s, d).astype(jnp.bfloat16)
    ones = jnp.ones((b * h, s, 1), jnp.bfloat16)
    zeros = jnp.zeros((b * h, s, d - 1), jnp.bfloat16)
    v_aug = jnp.concatenate([v, ones, zeros], axis=-1)

    grid = (b * h, s // BQ)
    out = pl.pallas_call(
        functools.partial(_flash_kernel, bq=BQ, bk=BK, d=d),
        grid=grid,
        in_specs=[
            pl.BlockSpec((1, BQ, d), lambda hh, i: (hh, i, 0)),
            pl.BlockSpec((1, s, d), lambda hh, i: (hh, 0, 0)),
            pl.BlockSpec((1, s, 2 * d), lambda hh, i: (hh, 0, 0)),
        ],
        out_specs=pl.BlockSpec((1, BQ, d), lambda hh, i: (hh, i, 0)),
        out_shape=jax.ShapeDtypeStruct((b * h, s, d), jnp.float32),
        compiler_params=pltpu.CompilerParams(
            dimension_semantics=("parallel", "arbitrary"),
        ),
    )(q, k, v_aug)
    return out.reshape(b, h, s, d)


# speedup vs baseline: 1.6209x; 1.6209x over previous
import functools

import jax
import jax.numpy as jnp
from jax.experimental import pallas as pl
from jax.experimental.pallas import tpu as pltpu

BQ = 256
NEG_INF = -1e30


def _flash_kernel(q_ref, k_ref, v_ref, o_ref, *, bq, d, s_len):
    qi = pl.program_id(1)
    q = q_ref[0]
    k = k_ref[0]
    v = v_ref[0]

    s = jax.lax.dot_general(
        q, k, (((1,), (1,)), ((), ())),
        preferred_element_type=jnp.float32)
    row = qi * bq + jax.lax.broadcasted_iota(jnp.int32, (bq, s_len), 0)
    col = jax.lax.broadcasted_iota(jnp.int32, (bq, s_len), 1)
    s = jnp.where(col <= row, s, NEG_INF)
    m = jnp.max(s, axis=-1)
    p = jnp.exp(s - m[:, None]).astype(jnp.bfloat16)
    acc = jax.lax.dot_general(
        p, v, (((1,), (0,)), ((), ())),
        preferred_element_type=jnp.float32)
    o_ref[0] = acc[:, :d] / acc[:, d:d + 1]


@jax.jit
def kernel(query, key, value):
    b, h, s, d = query.shape
    scale = d ** (-0.5)
    q = (query.reshape(b * h, s, d) * scale).astype(jnp.bfloat16)
    k = key.reshape(b * h, s, d).astype(jnp.bfloat16)
    v = value.reshape(b * h, s, d).astype(jnp.bfloat16)
    ones = jnp.ones((b * h, s, 1), jnp.bfloat16)
    zeros = jnp.zeros((b * h, s, d - 1), jnp.bfloat16)
    v_aug = jnp.concatenate([v, ones, zeros], axis=-1)

    grid = (b * h, s // BQ)
    out = pl.pallas_call(
        functools.partial(_flash_kernel, bq=BQ, d=d, s_len=s),
        grid=grid,
        in_specs=[
            pl.BlockSpec((1, BQ, d), lambda hh, i: (hh, i, 0)),
            pl.BlockSpec((1, s, d), lambda hh, i: (hh, 0, 0)),
            pl.BlockSpec((1, s, 2 * d), lambda hh, i: (hh, 0, 0)),
        ],
        out_specs=pl.BlockSpec((1, BQ, d), lambda hh, i: (hh, i, 0)),
        out_shape=jax.ShapeDtypeStruct((b * h, s, d), jnp.float32),
        compiler_params=pltpu.CompilerParams(
            dimension_semantics=("parallel", "arbitrary"),
        ),
    )(q, k, v_aug)
    return out.reshape(b, h, s, d)


# speedup vs baseline: 1.8059x; 1.1141x over previous
import functools

import jax
import jax.numpy as jnp
from jax.experimental import pallas as pl
from jax.experimental.pallas import tpu as pltpu

BQ = 512
NEG_INF = -1e30


def _band_kernel(q_ref, k_ref, v_ref, o_ref, *, band, bq, d, w):
    q = q_ref[0]
    k = k_ref[0]
    v = v_ref[0]

    s = jax.lax.dot_general(
        q, k, (((1,), (1,)), ((), ())),
        preferred_element_type=jnp.float32)
    row = band * bq + jax.lax.broadcasted_iota(jnp.int32, (bq, w), 0)
    col = jax.lax.broadcasted_iota(jnp.int32, (bq, w), 1)
    s = jnp.where(col <= row, s, NEG_INF)
    m = jnp.max(s, axis=-1)
    p = jnp.exp(s - m[:, None]).astype(jnp.bfloat16)
    acc = jax.lax.dot_general(
        p, v, (((1,), (0,)), ((), ())),
        preferred_element_type=jnp.float32)
    o_ref[0] = acc[:, :d] / acc[:, d:d + 1]


@jax.jit
def kernel(query, key, value):
    b, h, s, d = query.shape
    scale = d ** (-0.5)
    q = (query.reshape(b * h, s, d) * scale).astype(jnp.bfloat16)
    k = key.reshape(b * h, s, d).astype(jnp.bfloat16)
    v = value.reshape(b * h, s, d).astype(jnp.bfloat16)
    ones = jnp.ones((b * h, s, 1), jnp.bfloat16)
    zeros = jnp.zeros((b * h, s, d - 1), jnp.bfloat16)
    v_aug = jnp.concatenate([v, ones, zeros], axis=-1)

    n_bands = s // BQ
    outs = []
    for band in range(n_bands):
        w = (band + 1) * BQ
        out = pl.pallas_call(
            functools.partial(_band_kernel, band=band, bq=BQ, d=d, w=w),
            grid=(b * h,),
            in_specs=[
                pl.BlockSpec((1, BQ, d), lambda hh, _band=band: (hh, _band, 0)),
                pl.BlockSpec((1, w, d), lambda hh: (hh, 0, 0)),
                pl.BlockSpec((1, w, 2 * d), lambda hh: (hh, 0, 0)),
            ],
            out_specs=pl.BlockSpec((1, BQ, d), lambda hh: (hh, 0, 0)),
            out_shape=jax.ShapeDtypeStruct((b * h, BQ, d), jnp.float32),
            compiler_params=pltpu.CompilerParams(
                dimension_semantics=("parallel",),
            ),
        )(q, k, v_aug)
        outs.append(out)
    out = jnp.concatenate(outs, axis=1)
    return out.reshape(b, h, s, d)


# speedup vs baseline: 1.9302x; 1.0688x over previous
import functools

import jax
import jax.numpy as jnp
from jax.experimental import pallas as pl
from jax.experimental.pallas import tpu as pltpu

BQ = 512
NEG_INF = -1e30


def _band_kernel(q_ref, k_ref, v_ref, _, o_ref, *, band, bq, d, w, scale):
    q = (q_ref[0] * scale).astype(jnp.bfloat16)
    k = k_ref[0]
    v = v_ref[0]

    s = jax.lax.dot_general(
        q, k, (((1,), (1,)), ((), ())),
        preferred_element_type=jnp.float32)
    row = band * bq + jax.lax.broadcasted_iota(jnp.int32, (bq, w), 0)
    col = jax.lax.broadcasted_iota(jnp.int32, (bq, w), 1)
    s = jnp.where(col <= row, s, NEG_INF)
    m = jnp.max(s, axis=-1)
    p = jnp.exp(s - m[:, None]).astype(jnp.bfloat16)
    acc = jax.lax.dot_general(
        p, v, (((1,), (0,)), ((), ())),
        preferred_element_type=jnp.float32)
    o_ref[0] = acc[:, :d] / acc[:, d:d + 1]


@jax.jit
def kernel(query, key, value):
    b, h, s, d = query.shape
    scale = d ** (-0.5)
    q = query.reshape(b * h, s, d)
    k = key.reshape(b * h, s, d).astype(jnp.bfloat16)
    v = value.reshape(b * h, s, d).astype(jnp.bfloat16)
    ones = jnp.ones((b * h, s, 1), jnp.bfloat16)
    zeros = jnp.zeros((b * h, s, d - 1), jnp.bfloat16)
    v_aug = jnp.concatenate([v, ones, zeros], axis=-1)

    n_bands = s // BQ
    out = jnp.zeros((b * h, s, d), jnp.float32)
    for band in range(n_bands):
        w = (band + 1) * BQ
        out = pl.pallas_call(
            functools.partial(_band_kernel, band=band, bq=BQ, d=d, w=w,
                              scale=scale),
            grid=(b * h,),
            in_specs=[
                pl.BlockSpec((1, BQ, d), lambda hh, _band=band: (hh, _band, 0)),
                pl.BlockSpec((1, w, d), lambda hh: (hh, 0, 0)),
                pl.BlockSpec((1, w, 2 * d), lambda hh: (hh, 0, 0)),
                pl.BlockSpec(memory_space=pl.ANY),
            ],
            out_specs=pl.BlockSpec((1, BQ, d),
                                   lambda hh, _band=band: (hh, _band, 0)),
            out_shape=jax.ShapeDtypeStruct((b * h, s, d), jnp.float32),
            input_output_aliases={3: 0},
            compiler_params=pltpu.CompilerParams(
                dimension_semantics=("parallel",),
            ),
        )(q, k, v_aug, out)
    return out.reshape(b, h, s, d)


# speedup vs baseline: 1.9362x; 1.0031x over previous
import functools

import jax
import jax.numpy as jnp
from jax.experimental import pallas as pl
from jax.experimental.pallas import tpu as pltpu

BQ = 512
NEG_INF = -1e30


def _band_kernel(q_ref, k_ref, v_ref, _, o_ref, *, band, bq, d, w, scale):
    q = (q_ref[0] * scale).astype(jnp.bfloat16)
    wb = w - bq

    kd = k_ref[0, pl.ds(wb, bq), :]
    sd = jax.lax.dot_general(
        q, kd, (((1,), (1,)), ((), ())),
        preferred_element_type=jnp.float32)
    row = jax.lax.broadcasted_iota(jnp.int32, (bq, bq), 0)
    col = jax.lax.broadcasted_iota(jnp.int32, (bq, bq), 1)
    tri = col <= row
    sd = jnp.where(tri, sd, NEG_INF)
    m = jnp.max(sd, axis=-1)

    if wb > 0:
        kb = k_ref[0, pl.ds(0, wb), :]
        sb = jax.lax.dot_general(
            q, kb, (((1,), (1,)), ((), ())),
            preferred_element_type=jnp.float32)
        m = jnp.maximum(m, jnp.max(sb, axis=-1))

    pd = jnp.exp(sd - m[:, None]).astype(jnp.bfloat16)
    vd = v_ref[0, pl.ds(wb, bq), :]
    acc = jax.lax.dot_general(
        pd, vd, (((1,), (0,)), ((), ())),
        preferred_element_type=jnp.float32)

    if wb > 0:
        pb = jnp.exp(sb - m[:, None]).astype(jnp.bfloat16)
        vb = v_ref[0, pl.ds(0, wb), :]
        acc = acc + jax.lax.dot_general(
            pb, vb, (((1,), (0,)), ((), ())),
            preferred_element_type=jnp.float32)

    o_ref[0] = acc[:, :d] / acc[:, d:d + 1]


@jax.jit
def kernel(query, key, value):
    b, h, s, d = query.shape
    scale = d ** (-0.5)
    q = query.reshape(b * h, s, d)
    k = key.reshape(b * h, s, d).astype(jnp.bfloat16)
    v = value.reshape(b * h, s, d).astype(jnp.bfloat16)
    ones = jnp.ones((b * h, s, 1), jnp.bfloat16)
    zeros = jnp.zeros((b * h, s, d - 1), jnp.bfloat16)
    v_aug = jnp.concatenate([v, ones, zeros], axis=-1)

    n_bands = s // BQ
    out = jnp.zeros((b * h, s, d), jnp.float32)
    for band in range(n_bands):
        w = (band + 1) * BQ
        out = pl.pallas_call(
            functools.partial(_band_kernel, band=band, bq=BQ, d=d, w=w,
                              scale=scale),
            grid=(b * h,),
            in_specs=[
                pl.BlockSpec((1, BQ, d), lambda hh, _band=band: (hh, _band, 0)),
                pl.BlockSpec((1, w, d), lambda hh: (hh, 0, 0)),
                pl.BlockSpec((1, w, 2 * d), lambda hh: (hh, 0, 0)),
                pl.BlockSpec(memory_space=pl.ANY),
            ],
            out_specs=pl.BlockSpec((1, BQ, d),
                                   lambda hh, _band=band: (hh, _band, 0)),
            out_shape=jax.ShapeDtypeStruct((b * h, s, d), jnp.float32),
            input_output_aliases={3: 0},
            compiler_params=pltpu.CompilerParams(
                dimension_semantics=("parallel",),
            ),
        )(q, k, v_aug, out)
    return out.reshape(b, h, s, d)


# speedup vs baseline: 1.9887x; 1.0271x over previous
import functools

import jax
import jax.numpy as jnp
from jax.experimental import pallas as pl
from jax.experimental.pallas import tpu as pltpu

BQ = 512
NEG_INF = -1e30


def _band_body(q_ref, k_ref, v_ref, o_ref, *, band, bq, d, w, scale):
    q = (q_ref[0] * scale).astype(jnp.bfloat16)
    wb = w - bq

    kd = k_ref[0, pl.ds(wb, bq), :].astype(jnp.bfloat16)
    sd = jax.lax.dot_general(
        q, kd, (((1,), (1,)), ((), ())),
        preferred_element_type=jnp.float32)
    row = jax.lax.broadcasted_iota(jnp.int32, (bq, bq), 0)
    col = jax.lax.broadcasted_iota(jnp.int32, (bq, bq), 1)
    sd = jnp.where(col <= row, sd, NEG_INF)
    m = jnp.max(sd, axis=-1)

    if wb > 0:
        kb = k_ref[0, pl.ds(0, wb), :].astype(jnp.bfloat16)
        sb = jax.lax.dot_general(
            q, kb, (((1,), (1,)), ((), ())),
            preferred_element_type=jnp.float32)
        m = jnp.maximum(m, jnp.max(sb, axis=-1))

    pd = jnp.exp(sd - m[:, None]).astype(jnp.bfloat16)
    vd = jnp.concatenate(
        [v_ref[0, pl.ds(wb, bq), :].astype(jnp.bfloat16),
         jnp.ones((bq, d), jnp.bfloat16)], axis=-1)
    acc = jax.lax.dot_general(
        pd, vd, (((1,), (0,)), ((), ())),
        preferred_element_type=jnp.float32)

    if wb > 0:
        pb = jnp.exp(sb - m[:, None]).astype(jnp.bfloat16)
        vb = jnp.concatenate(
            [v_ref[0, pl.ds(0, wb), :].astype(jnp.bfloat16),
             jnp.ones((wb, d), jnp.bfloat16)], axis=-1)
        acc = acc + jax.lax.dot_general(
            pb, vb, (((1,), (0,)), ((), ())),
            preferred_element_type=jnp.float32)

    o_ref[0] = acc[:, :d] / acc[:, d:d + 1]


def _band_kernel_first(q_ref, k_ref, v_ref, o_ref, **kw):
    _band_body(q_ref, k_ref, v_ref, o_ref, **kw)


def _band_kernel_chained(q_ref, k_ref, v_ref, _, o_ref, **kw):
    _band_body(q_ref, k_ref, v_ref, o_ref, **kw)


@jax.jit
def kernel(query, key, value):
    b, h, s, d = query.shape
    scale = d ** (-0.5)
    q = query.reshape(b * h, s, d)
    k = key.reshape(b * h, s, d)
    v = value.reshape(b * h, s, d)

    n_bands = s // BQ
    out = None
    for band in range(n_bands):
        w = (band + 1) * BQ
        body = functools.partial(
            _band_kernel_first if band == 0 else _band_kernel_chained,
            band=band, bq=BQ, d=d, w=w, scale=scale)
        in_specs = [
            pl.BlockSpec((1, BQ, d), lambda hh, _band=band: (hh, _band, 0)),
            pl.BlockSpec((1, w, d), lambda hh: (hh, 0, 0)),
            pl.BlockSpec((1, w, d), lambda hh: (hh, 0, 0)),
        ]
        operands = [q, k, v]
        aliases = {}
        if band > 0:
            in_specs.append(pl.BlockSpec(memory_space=pl.ANY))
            operands.append(out)
            aliases = {3: 0}
        out = pl.pallas_call(
            body,
            grid=(b * h,),
            in_specs=in_specs,
            out_specs=pl.BlockSpec((1, BQ, d),
                                   lambda hh, _band=band: (hh, _band, 0)),
            out_shape=jax.ShapeDtypeStruct((b * h, s, d), jnp.float32),
            input_output_aliases=aliases,
            compiler_params=pltpu.CompilerParams(
                dimension_semantics=("parallel",),
            ),
        )(*operands)
    return out.reshape(b, h, s, d)


# speedup vs baseline: 2.6150x; 1.3149x over previous
import functools

import jax
import jax.numpy as jnp
from jax.experimental import pallas as pl
from jax.experimental.pallas import tpu as pltpu

BQ = 512
NEG_INF = -1e30


def _head_kernel(q_ref, k_ref, v_ref, o_ref, kbf_ref, vaug_ref,
                 *, bq, d, s_len, scale):
    kbf_ref[:, :] = k_ref[0].astype(jnp.bfloat16)
    vaug_ref[:, :d] = v_ref[0].astype(jnp.bfloat16)
    vaug_ref[:, d:] = jnp.ones((s_len, d), jnp.bfloat16)

    row = jax.lax.broadcasted_iota(jnp.int32, (bq, bq), 0)
    col = jax.lax.broadcasted_iota(jnp.int32, (bq, bq), 1)
    tri = col <= row

    for band in range(s_len // bq):
        wb = band * bq
        q = (q_ref[0, pl.ds(wb, bq), :] * scale).astype(jnp.bfloat16)

        kd = kbf_ref[pl.ds(wb, bq), :]
        sd = jax.lax.dot_general(
            q, kd, (((1,), (1,)), ((), ())),
            preferred_element_type=jnp.float32)
        sd = jnp.where(tri, sd, NEG_INF)
        m = jnp.max(sd, axis=-1)

        if wb > 0:
            kb = kbf_ref[pl.ds(0, wb), :]
            sb = jax.lax.dot_general(
                q, kb, (((1,), (1,)), ((), ())),
                preferred_element_type=jnp.float32)
            m = jnp.maximum(m, jnp.max(sb, axis=-1))

        pd = jnp.exp(sd - m[:, None]).astype(jnp.bfloat16)
        acc = jax.lax.dot_general(
            pd, vaug_ref[pl.ds(wb, bq), :], (((1,), (0,)), ((), ())),
            preferred_element_type=jnp.float32)

        if wb > 0:
            pb = jnp.exp(sb - m[:, None]).astype(jnp.bfloat16)
            acc = acc + jax.lax.dot_general(
                pb, vaug_ref[pl.ds(0, wb), :], (((1,), (0,)), ((), ())),
                preferred_element_type=jnp.float32)

        o_ref[0, pl.ds(wb, bq), :] = acc[:, :d] / acc[:, d:d + 1]


@jax.jit
def kernel(query, key, value):
    b, h, s, d = query.shape
    scale = d ** (-0.5)
    q = query.reshape(b * h, s, d)
    k = key.reshape(b * h, s, d)
    v = value.reshape(b * h, s, d)

    out = pl.pallas_call(
        functools.partial(_head_kernel, bq=BQ, d=d, s_len=s, scale=scale),
        grid=(b * h,),
        in_specs=[
            pl.BlockSpec((1, s, d), lambda hh: (hh, 0, 0)),
            pl.BlockSpec((1, s, d), lambda hh: (hh, 0, 0)),
            pl.BlockSpec((1, s, d), lambda hh: (hh, 0, 0)),
        ],
        out_specs=pl.BlockSpec((1, s, d), lambda hh: (hh, 0, 0)),
        out_shape=jax.ShapeDtypeStruct((b * h, s, d), jnp.float32),
        scratch_shapes=[
            pltpu.VMEM((s, d), jnp.bfloat16),
            pltpu.VMEM((s, 2 * d), jnp.bfloat16),
        ],
        compiler_params=pltpu.CompilerParams(
            dimension_semantics=("parallel",),
        ),
    )(q, k, v)
    return out.reshape(b, h, s, d)


# speedup vs baseline: 2.6516x; 1.0140x over previous
import functools

import jax
import jax.numpy as jnp
from jax.experimental import pallas as pl
from jax.experimental.pallas import tpu as pltpu

BQ = 512
NEG_INF = -1e30


def _head_kernel(q_ref, k_ref, v_ref, o_ref, kbf_ref, vaug_ref,
                 *, bq, d, s_len, scale):
    scale2 = scale * 1.4426950408889634
    kbf_ref[:, :] = k_ref[0].astype(jnp.bfloat16)
    vaug_ref[:, :d] = v_ref[0].astype(jnp.bfloat16)
    vaug_ref[:, d:] = jnp.ones((s_len, d), jnp.bfloat16)

    row = jax.lax.broadcasted_iota(jnp.int32, (bq, bq), 0)
    col = jax.lax.broadcasted_iota(jnp.int32, (bq, bq), 1)
    tri = col <= row

    for band in range(s_len // bq):
        wb = band * bq
        q = (q_ref[0, pl.ds(wb, bq), :] * scale2).astype(jnp.bfloat16)

        kd = kbf_ref[pl.ds(wb, bq), :]
        sd = jax.lax.dot_general(
            q, kd, (((1,), (1,)), ((), ())),
            preferred_element_type=jnp.float32)
        sd = jnp.where(tri, sd, NEG_INF)
        m = jnp.max(sd, axis=-1)

        if wb > 0:
            kb = kbf_ref[pl.ds(0, wb), :]
            sb = jax.lax.dot_general(
                q, kb, (((1,), (1,)), ((), ())),
                preferred_element_type=jnp.float32)
            m = jnp.maximum(m, jnp.max(sb, axis=-1))

        pd = jnp.exp2(sd - m[:, None]).astype(jnp.bfloat16)
        acc = jax.lax.dot_general(
            pd, vaug_ref[pl.ds(wb, bq), :], (((1,), (0,)), ((), ())),
            preferred_element_type=jnp.float32)

        if wb > 0:
            pb = jnp.exp2(sb - m[:, None]).astype(jnp.bfloat16)
            acc = acc + jax.lax.dot_general(
                pb, vaug_ref[pl.ds(0, wb), :], (((1,), (0,)), ((), ())),
                preferred_element_type=jnp.float32)

        o_ref[0, pl.ds(wb, bq), :] = acc[:, :d] / acc[:, d:d + 1]


@jax.jit
def kernel(query, key, value):
    b, h, s, d = query.shape
    scale = d ** (-0.5)
    q = query.reshape(b * h, s, d)
    k = key.reshape(b * h, s, d)
    v = value.reshape(b * h, s, d)

    out = pl.pallas_call(
        functools.partial(_head_kernel, bq=BQ, d=d, s_len=s, scale=scale),
        grid=(b * h,),
        in_specs=[
            pl.BlockSpec((1, s, d), lambda hh: (hh, 0, 0)),
            pl.BlockSpec((1, s, d), lambda hh: (hh, 0, 0)),
            pl.BlockSpec((1, s, d), lambda hh: (hh, 0, 0)),
        ],
        out_specs=pl.BlockSpec((1, s, d), lambda hh: (hh, 0, 0)),
        out_shape=jax.ShapeDtypeStruct((b * h, s, d), jnp.float32),
        scratch_shapes=[
            pltpu.VMEM((s, d), jnp.bfloat16),
            pltpu.VMEM((s, 2 * d), jnp.bfloat16),
        ],
        compiler_params=pltpu.CompilerParams(
            dimension_semantics=("parallel",),
        ),
    )(q, k, v)
    return out.reshape(b, h, s, d)
